# speedup vs baseline: 3.1847x; 3.1847x over previous
import functools

import jax
import jax.numpy as jnp
from jax import lax
from jax.experimental import pallas as pl
from jax.experimental.pallas import tpu as pltpu
from jax.experimental.pallas import tpu_sc as plsc

D = 128
TOTAL = 4096 * 50

NC = 2
NS = 16
NW = NC * NS
B_PER_W = TOTAL // NW
CHUNK = 400
NCHUNK = B_PER_W // CHUNK

_mesh = plsc.VectorSubcoreMesh(core_axis_name="c", subcore_axis_name="s")


@functools.partial(
    pl.kernel,
    mesh=_mesh,
    out_type=jax.ShapeDtypeStruct((TOTAL, D), jnp.float32),
    scratch_types=[
        pltpu.VMEM((CHUNK,), jnp.int32),
        pltpu.VMEM((CHUNK, D), jnp.float32),
        pltpu.SemaphoreType.DMA,
    ],
)
def _gather_kernel(idx_hbm, table_hbm, out_hbm, idx_v, rows_v, sem):
    wid = lax.axis_index("s") * NC + lax.axis_index("c")
    base = wid * B_PER_W

    def body(j, carry):
        off = base + j * CHUNK
        pltpu.sync_copy(idx_hbm.at[pl.ds(off, CHUNK)], idx_v)
        pltpu.async_copy(table_hbm.at[idx_v], rows_v, sem).wait()
        pltpu.sync_copy(rows_v, out_hbm.at[pl.ds(off, CHUNK)])
        return carry

    lax.fori_loop(0, NCHUNK, body, 0)


def kernel(input_indices, token_embedding_table):
    idx = input_indices.reshape(-1).astype(jnp.int32)
    out = _gather_kernel(idx, token_embedding_table)
    return out.reshape(input_indices.shape + (D,))


# speedup vs baseline: 3.3371x; 1.0478x over previous
import functools

import jax
import jax.numpy as jnp
from jax import lax
from jax.experimental import pallas as pl
from jax.experimental.pallas import tpu as pltpu
from jax.experimental.pallas import tpu_sc as plsc

D = 128
TOTAL = 4096 * 50

NC = 2
NS = 16
NW = NC * NS
B_PER_W = TOTAL // NW
CHUNK = 400
NCHUNK = B_PER_W // CHUNK

_mesh = plsc.VectorSubcoreMesh(core_axis_name="c", subcore_axis_name="s")


@functools.partial(
    pl.kernel,
    mesh=_mesh,
    out_type=jax.ShapeDtypeStruct((TOTAL, D), jnp.float32),
    scratch_types=[
        pltpu.VMEM((CHUNK,), jnp.int32),
        pltpu.VMEM((CHUNK, D), jnp.float32),
        pltpu.VMEM((CHUNK,), jnp.int32),
        pltpu.VMEM((CHUNK, D), jnp.float32),
        pltpu.SemaphoreType.DMA,
        pltpu.SemaphoreType.DMA,
    ],
)
def _gather_kernel(idx_hbm, table_hbm, out_hbm,
                   idx_a, rows_a, idx_b, rows_b, sem_a, sem_b):
    wid = lax.axis_index("s") * NC + lax.axis_index("c")
    base = wid * B_PER_W

    def start(j, idx_v, rows_v, sem):
        off = base + j * CHUNK
        pltpu.sync_copy(idx_hbm.at[pl.ds(off, CHUNK)], idx_v)
        pltpu.async_copy(table_hbm.at[idx_v], rows_v, sem)

    def finish(j, idx_v, rows_v, sem):
        off = base + j * CHUNK
        pltpu.make_async_copy(table_hbm.at[idx_v], rows_v, sem).wait()
        pltpu.sync_copy(rows_v, out_hbm.at[pl.ds(off, CHUNK)])

    start(0, idx_a, rows_a, sem_a)
    start(1, idx_b, rows_b, sem_b)

    def body(t, carry):
        j = 2 * t
        finish(j, idx_a, rows_a, sem_a)
        start(j + 2, idx_a, rows_a, sem_a)
        finish(j + 1, idx_b, rows_b, sem_b)
        start(j + 3, idx_b, rows_b, sem_b)
        return carry

    lax.fori_loop(0, NCHUNK // 2 - 1, body, 0)
    finish(NCHUNK - 2, idx_a, rows_a, sem_a)
    finish(NCHUNK - 1, idx_b, rows_b, sem_b)


def kernel(input_indices, token_embedding_table):
    idx = input_indices.reshape(-1).astype(jnp.int32)
    out = _gather_kernel(idx, token_embedding_table)
    return out.reshape(input_indices.shape + (D,))


# speedup vs baseline: 3.3556x; 1.0056x over previous
import functools

import jax
import jax.numpy as jnp
from jax import lax
from jax.experimental import pallas as pl
from jax.experimental.pallas import tpu as pltpu
from jax.experimental.pallas import tpu_sc as plsc

D = 128
TOTAL = 4096 * 50

NC = 2
NS = 16
NW = NC * NS
B_PER_W = TOTAL // NW
CHUNK = 400
NCHUNK = B_PER_W // CHUNK

_mesh = plsc.VectorSubcoreMesh(core_axis_name="c", subcore_axis_name="s")


@functools.partial(
    pl.kernel,
    mesh=_mesh,
    out_type=jax.ShapeDtypeStruct((TOTAL, D), jnp.float32),
    scratch_types=[
        pltpu.VMEM((B_PER_W,), jnp.int32),
        pltpu.VMEM((CHUNK, D), jnp.float32),
        pltpu.VMEM((CHUNK, D), jnp.float32),
        pltpu.SemaphoreType.DMA,
        pltpu.SemaphoreType.DMA,
    ],
)
def _gather_kernel(idx_hbm, table_hbm, out_hbm,
                   idx_v, rows_a, rows_b, sem_a, sem_b):
    wid = lax.axis_index("s") * NC + lax.axis_index("c")
    base = wid * B_PER_W
    pltpu.sync_copy(idx_hbm.at[pl.ds(base, B_PER_W)], idx_v)

    def start(j, rows_v, sem):
        pltpu.async_copy(table_hbm.at[idx_v.at[pl.ds(j * CHUNK, CHUNK)]],
                         rows_v, sem)

    def finish(j, rows_v, sem):
        pltpu.make_async_copy(table_hbm.at[idx_v.at[pl.ds(j * CHUNK, CHUNK)]],
                              rows_v, sem).wait()
        pltpu.sync_copy(rows_v, out_hbm.at[pl.ds(base + j * CHUNK, CHUNK)])

    start(0, rows_a, sem_a)
    start(1, rows_b, sem_b)

    def body(t, carry):
        j = 2 * t
        finish(j, rows_a, sem_a)
        start(j + 2, rows_a, sem_a)
        finish(j + 1, rows_b, sem_b)
        start(j + 3, rows_b, sem_b)
        return carry

    lax.fori_loop(0, NCHUNK // 2 - 1, body, 0)
    finish(NCHUNK - 2, rows_a, sem_a)
    finish(NCHUNK - 1, rows_b, sem_b)


def kernel(input_indices, token_embedding_table):
    idx = input_indices.reshape(-1).astype(jnp.int32)
    out = _gather_kernel(idx, token_embedding_table)
    return out.reshape(input_indices.shape + (D,))


# speedup vs baseline: 5.6042x; 1.6701x over previous
import functools

import jax
import jax.numpy as jnp
from jax import lax
from jax.experimental import pallas as pl
from jax.experimental.pallas import tpu as pltpu
from jax.experimental.pallas import tpu_sc as plsc

B = 4096
H = 50
D = 128
TOTAL = B * H

NC = 2
NS = 16
NW = NC * NS
B_PER_W = B // NW
IDX_PER_W = B_PER_W * H
BCHUNK = 8
CHUNK = BCHUNK * H
NCHUNK = B_PER_W // BCHUNK

_mesh = plsc.VectorSubcoreMesh(core_axis_name="c", subcore_axis_name="s")


@functools.partial(
    pl.kernel,
    mesh=_mesh,
    out_type=jax.ShapeDtypeStruct((B, H, D), jnp.float32),
    scratch_types=[
        pltpu.VMEM((IDX_PER_W,), jnp.int32),
        pltpu.VMEM((CHUNK, D), jnp.float32),
        pltpu.VMEM((CHUNK, D), jnp.float32),
        pltpu.SemaphoreType.DMA,
        pltpu.SemaphoreType.DMA,
        pltpu.SemaphoreType.DMA,
        pltpu.SemaphoreType.DMA,
    ],
)
def _gather_kernel(idx_hbm, table_hbm, out_hbm,
                   idx_v, rows_a, rows_b, gsem_a, gsem_b, ssem_a, ssem_b):
    wid = lax.axis_index("s") * NC + lax.axis_index("c")
    bbase = wid * B_PER_W
    pltpu.sync_copy(idx_hbm.at[pl.ds(wid * IDX_PER_W, IDX_PER_W)], idx_v)

    def start(j, rows_v, gsem):
        pltpu.async_copy(table_hbm.at[idx_v.at[pl.ds(j * CHUNK, CHUNK)]],
                         rows_v, gsem)

    def store(j, rows_v, gsem, ssem):
        pltpu.make_async_copy(table_hbm.at[idx_v.at[pl.ds(j * CHUNK, CHUNK)]],
                              rows_v, gsem).wait()
        for i in range(BCHUNK):
            pltpu.async_copy(rows_v.at[pl.ds(i * H, H)],
                             out_hbm.at[bbase + j * BCHUNK + i], ssem)

    def drain(rows_v, ssem):
        for i in range(BCHUNK):
            pltpu.make_async_copy(rows_v.at[pl.ds(i * H, H)],
                                  out_hbm.at[bbase + i], ssem).wait()

    start(0, rows_a, gsem_a)
    start(1, rows_b, gsem_b)

    def body(t, carry):
        j = 2 * t
        store(j, rows_a, gsem_a, ssem_a)
        store(j + 1, rows_b, gsem_b, ssem_b)
        drain(rows_a, ssem_a)
        start(j + 2, rows_a, gsem_a)
        drain(rows_b, ssem_b)
        start(j + 3, rows_b, gsem_b)
        return carry

    lax.fori_loop(0, NCHUNK // 2 - 1, body, 0)
    store(NCHUNK - 2, rows_a, gsem_a, ssem_a)
    store(NCHUNK - 1, rows_b, gsem_b, ssem_b)
    drain(rows_a, ssem_a)
    drain(rows_b, ssem_b)


def kernel(input_indices, token_embedding_table):
    idx = input_indices.reshape(-1).astype(jnp.int32)
    return _gather_kernel(idx, token_embedding_table)


# speedup vs baseline: 5.6424x; 1.0068x over previous
import functools

import jax
import jax.numpy as jnp
from jax import lax
from jax.experimental import pallas as pl
from jax.experimental.pallas import tpu as pltpu
from jax.experimental.pallas import tpu_sc as plsc

B = 4096
H = 50
D = 128
TOTAL = B * H

NC = 2
NS = 16
NW = NC * NS
B_PER_W = B // NW
IDX_PER_W = B_PER_W * H
BCHUNK = 8
CHUNK = BCHUNK * H
NCHUNK = B_PER_W // BCHUNK

_mesh = plsc.VectorSubcoreMesh(core_axis_name="c", subcore_axis_name="s")


@functools.partial(
    pl.kernel,
    mesh=_mesh,
    out_type=jax.ShapeDtypeStruct((B, H, D), jnp.float32),
    compiler_params=pltpu.CompilerParams(use_tc_tiling_on_sc=True),
    scratch_types=[
        pltpu.VMEM((IDX_PER_W,), jnp.int32),
        pltpu.VMEM((CHUNK, D), jnp.float32),
        pltpu.VMEM((CHUNK, D), jnp.float32),
        pltpu.SemaphoreType.DMA,
        pltpu.SemaphoreType.DMA,
        pltpu.SemaphoreType.DMA,
        pltpu.SemaphoreType.DMA,
    ],
)
def _gather_kernel(idx_hbm, table_hbm, out_hbm,
                   idx_v, rows_a, rows_b, gsem_a, gsem_b, ssem_a, ssem_b):
    wid = lax.axis_index("s") * NC + lax.axis_index("c")
    bbase = wid * B_PER_W
    pltpu.sync_copy(idx_hbm.at[pl.ds(wid * IDX_PER_W, IDX_PER_W)], idx_v)

    def start(j, rows_v, gsem):
        pltpu.async_copy(table_hbm.at[idx_v.at[pl.ds(j * CHUNK, CHUNK)]],
                         rows_v, gsem)

    def store(j, rows_v, gsem, ssem):
        pltpu.make_async_copy(table_hbm.at[idx_v.at[pl.ds(j * CHUNK, CHUNK)]],
                              rows_v, gsem).wait()
        for i in range(BCHUNK):
            pltpu.async_copy(rows_v.at[pl.ds(i * H, H)],
                             out_hbm.at[bbase + j * BCHUNK + i], ssem)

    def drain(rows_v, ssem):
        for i in range(BCHUNK):
            pltpu.make_async_copy(rows_v.at[pl.ds(i * H, H)],
                                  out_hbm.at[bbase + i], ssem).wait()

    start(0, rows_a, gsem_a)
    start(1, rows_b, gsem_b)

    def body(t, carry):
        j = 2 * t
        store(j, rows_a, gsem_a, ssem_a)
        store(j + 1, rows_b, gsem_b, ssem_b)
        drain(rows_a, ssem_a)
        start(j + 2, rows_a, gsem_a)
        drain(rows_b, ssem_b)
        start(j + 3, rows_b, gsem_b)
        return carry

    lax.fori_loop(0, NCHUNK // 2 - 1, body, 0)
    store(NCHUNK - 2, rows_a, gsem_a, ssem_a)
    store(NCHUNK - 1, rows_b, gsem_b, ssem_b)
    drain(rows_a, ssem_a)
    drain(rows_b, ssem_b)


def kernel(input_indices, token_embedding_table):
    idx = input_indices.reshape(-1).astype(jnp.int32)
    return _gather_kernel(idx, token_embedding_table)


# speedup vs baseline: 9.8136x; 1.7393x over previous
import functools

import jax
import jax.numpy as jnp
from jax import lax
from jax.experimental import pallas as pl
from jax.experimental.pallas import tpu as pltpu
from jax.experimental.pallas import tpu_sc as plsc

B = 4096
H = 50
D = 128
TOTAL = B * H

NC = 2
NS = 16
NW = NC * NS
B_PER_W = TOTAL // NW
NBUF = 4
CHUNK = 200
NCHUNK = B_PER_W // CHUNK

_mesh = plsc.VectorSubcoreMesh(core_axis_name="c", subcore_axis_name="s")


@functools.partial(
    pl.kernel,
    mesh=_mesh,
    out_type=jax.ShapeDtypeStruct((TOTAL, D), jnp.float32),
    scratch_types=[
        pltpu.VMEM((B_PER_W,), jnp.int32),
        *[pltpu.VMEM((CHUNK, D), jnp.float32) for _ in range(NBUF)],
        *[pltpu.SemaphoreType.DMA for _ in range(2 * NBUF)],
    ],
)
def _gather_kernel(idx_hbm, table_hbm, out_hbm, idx_v, *bufs_and_sems):
    bufs = bufs_and_sems[:NBUF]
    gsems = bufs_and_sems[NBUF:2 * NBUF]
    wsems = bufs_and_sems[2 * NBUF:]
    wid = lax.axis_index("s") * NC + lax.axis_index("c")
    base = wid * B_PER_W
    pltpu.sync_copy(idx_hbm.at[pl.ds(base, B_PER_W)], idx_v)

    def gather(j, b):
        pltpu.async_copy(table_hbm.at[idx_v.at[pl.ds(j * CHUNK, CHUNK)]],
                         bufs[b], gsems[b])

    def wait_gather(j, b):
        pltpu.make_async_copy(table_hbm.at[idx_v.at[pl.ds(j * CHUNK, CHUNK)]],
                              bufs[b], gsems[b]).wait()

    def write(j, b):
        pltpu.async_copy(bufs[b], out_hbm.at[pl.ds(base + j * CHUNK, CHUNK)],
                         wsems[b])

    def wait_write(j, b):
        pltpu.make_async_copy(bufs[b],
                              out_hbm.at[pl.ds(base + j * CHUNK, CHUNK)],
                              wsems[b]).wait()

    for b in range(NBUF):
        gather(b, b)

    def body(t, carry):
        j0 = NBUF * t
        for b in range(NBUF):
            wait_gather(j0 + b, b)
            write(j0 + b, b)
        for b in range(NBUF):
            wait_write(j0 + b, b)
            gather(j0 + NBUF + b, b)
        return carry

    lax.fori_loop(0, NCHUNK // NBUF - 1, body, 0)
    j0 = NCHUNK - NBUF
    for b in range(NBUF):
        wait_gather(j0 + b, b)
        write(j0 + b, b)
    for b in range(NBUF):
        wait_write(j0 + b, b)


def kernel(input_indices, token_embedding_table):
    idx = input_indices.T.reshape(-1).astype(jnp.int32)
    out = _gather_kernel(idx, token_embedding_table)
    return out.reshape(H, B, D).transpose(1, 0, 2)
